# baseline (speedup 1.0000x reference)
import jax
import jax.numpy as jnp
from jax import lax
from jax.experimental import pallas as pl
from jax.experimental.pallas import tpu as pltpu

_E = 16


def _gate_kernel(h_ref, w_ref, idx_ref, wt_ref):
    h = h_ref[...]
    w = w_ref[...]
    logits = lax.dot_general(h, w, (((1,), (1,)), ((), ())),
                             preferred_element_type=jnp.float32)
    lane = lax.broadcasted_iota(jnp.int32, logits.shape, 1)
    m1 = jnp.max(logits, axis=1, keepdims=True)
    i1 = jnp.min(jnp.where(logits == m1, lane, _E), axis=1, keepdims=True)
    masked = jnp.where(lane == i1, -jnp.inf, logits)
    m2 = jnp.max(masked, axis=1, keepdims=True)
    i2 = jnp.min(jnp.where(masked == m2, lane, _E), axis=1, keepdims=True)
    e2 = jnp.exp(m2 - m1)
    denom = 1.0 + e2
    idx_ref[...] = jnp.concatenate([i1, i2], axis=1)
    wt_ref[...] = jnp.concatenate([1.0 / denom, e2 / denom], axis=1)


def kernel(hidden_states, weight):
    bsz, seq_len, dim = hidden_states.shape
    h = hidden_states.reshape(-1, dim)
    tokens = h.shape[0]
    bt = 512
    grid = (tokens // bt,)
    idx, wt = pl.pallas_call(
        _gate_kernel,
        grid=grid,
        in_specs=[
            pl.BlockSpec((bt, dim), lambda i: (i, 0)),
            pl.BlockSpec((_E, dim), lambda i: (0, 0)),
        ],
        out_specs=[
            pl.BlockSpec((bt, 2), lambda i: (i, 0)),
            pl.BlockSpec((bt, 2), lambda i: (i, 0)),
        ],
        out_shape=[
            jax.ShapeDtypeStruct((tokens, 2), jnp.int32),
            jax.ShapeDtypeStruct((tokens, 2), jnp.float32),
        ],
        compiler_params=pltpu.CompilerParams(
            dimension_semantics=("parallel",)),
    )(h, weight)
    return (idx, wt, jnp.float32(0.0))


# speedup vs baseline: 1.2145x; 1.2145x over previous
import jax
import jax.numpy as jnp
from jax import lax
from jax.experimental import pallas as pl
from jax.experimental.pallas import tpu as pltpu

_E = 16


def _gate_kernel(h_ref, w_ref, idx_ref, wt_ref):
    h = h_ref[...]
    w = w_ref[...]
    logits = lax.dot_general(h, w, (((1,), (1,)), ((), ())),
                             preferred_element_type=jnp.float32)
    lane = lax.broadcasted_iota(jnp.int32, logits.shape, 1)
    m1 = jnp.max(logits, axis=1, keepdims=True)
    i1 = jnp.min(jnp.where(logits == m1, lane, _E), axis=1, keepdims=True)
    masked = jnp.where(lane == i1, -jnp.inf, logits)
    m2 = jnp.max(masked, axis=1, keepdims=True)
    i2 = jnp.min(jnp.where(masked == m2, lane, _E), axis=1, keepdims=True)
    e2 = jnp.exp(m2 - m1)
    denom = 1.0 + e2
    idx_ref[...] = jnp.concatenate([i1, i2], axis=1)
    wt_ref[...] = jnp.concatenate([1.0 / denom, e2 / denom], axis=1)


def kernel(hidden_states, weight):
    bsz, seq_len, dim = hidden_states.shape
    h = hidden_states.reshape(-1, dim)
    tokens = h.shape[0]
    bt = 2048
    grid = (tokens // bt,)
    idx, wt = pl.pallas_call(
        _gate_kernel,
        grid=grid,
        in_specs=[
            pl.BlockSpec((bt, dim), lambda i: (i, 0)),
            pl.BlockSpec((_E, dim), lambda i: (0, 0)),
        ],
        out_specs=[
            pl.BlockSpec((bt, 2), lambda i: (i, 0)),
            pl.BlockSpec((bt, 2), lambda i: (i, 0)),
        ],
        out_shape=[
            jax.ShapeDtypeStruct((tokens, 2), jnp.int32),
            jax.ShapeDtypeStruct((tokens, 2), jnp.float32),
        ],
        compiler_params=pltpu.CompilerParams(
            dimension_semantics=("parallel",)),
    )(h, weight)
    return (idx, wt, jnp.float32(0.0))
